# Initial kernel scaffold; baseline (speedup 1.0000x reference)
import jax
import jax.numpy as jnp
from jax.experimental import pallas as pl


def kernel(Fea_A_r, Fea_B_r, Fea_C_r, Fea_A_i, Fea_B_i, Fea_C_i):
    raise NotImplementedError("write your pallas kernel here")


# speedup vs baseline: 6.5196x; 6.5196x over previous
import jax
import jax.numpy as jnp
from jax import lax
from jax.experimental import pallas as pl
from jax.experimental.pallas import tpu as pltpu
from jax.experimental.pallas import tpu_sc as plsc
import functools

N_TOTAL = 16 * 192 * 56 * 56
NC, NS, L = 2, 16, 16
NW = NC * NS
PER_W = N_TOTAL // NW
CHUNK = 6144
CHUNKS = PER_W // CHUNK
VSTEPS = CHUNK // L

_mesh = plsc.VectorSubcoreMesh(
    core_axis_name="c", subcore_axis_name="s", num_cores=NC, num_subcores=NS
)


@functools.partial(
    pl.kernel,
    out_type=(
        jax.ShapeDtypeStruct((N_TOTAL,), jnp.float32),
        jax.ShapeDtypeStruct((N_TOTAL,), jnp.float32),
    ),
    mesh=_mesh,
    scratch_types=[
        pltpu.VMEM((CHUNK,), jnp.float32),
        pltpu.VMEM((CHUNK,), jnp.float32),
        pltpu.VMEM((CHUNK,), jnp.float32),
        pltpu.VMEM((CHUNK,), jnp.float32),
        pltpu.VMEM((CHUNK,), jnp.float32),
        pltpu.VMEM((CHUNK,), jnp.float32),
        pltpu.VMEM((CHUNK,), jnp.float32),
        pltpu.VMEM((CHUNK,), jnp.float32),
    ],
)
def _sc_max_fusion(ar_h, br_h, cr_h, ai_h, bi_h, ci_h, or_h, oi_h,
                   ar_v, br_v, cr_v, ai_v, bi_v, ci_v, orv, oiv):
    wid = lax.axis_index("s") * NC + lax.axis_index("c")
    base = wid * PER_W

    def chunk_body(k, _):
        off = pl.multiple_of(base + k * CHUNK, CHUNK)
        pltpu.sync_copy(ar_h.at[pl.ds(off, CHUNK)], ar_v)
        pltpu.sync_copy(br_h.at[pl.ds(off, CHUNK)], br_v)
        pltpu.sync_copy(cr_h.at[pl.ds(off, CHUNK)], cr_v)
        pltpu.sync_copy(ai_h.at[pl.ds(off, CHUNK)], ai_v)
        pltpu.sync_copy(bi_h.at[pl.ds(off, CHUNK)], bi_v)
        pltpu.sync_copy(ci_h.at[pl.ds(off, CHUNK)], ci_v)

        def vec_body(j, _):
            s = pl.ds(j * L, L)
            ra = ar_v[s]
            ia = ai_v[s]
            rb = br_v[s]
            ib = bi_v[s]
            rc = cr_v[s]
            ic = ci_v[s]
            ma = ra * ra + ia * ia
            mb = rb * rb + ib * ib
            mc = rc * rc + ic * ic
            b_wins = mb > ma
            r1 = jnp.where(b_wins, rb, ra)
            i1 = jnp.where(b_wins, ib, ia)
            m1 = jnp.maximum(ma, mb)
            c_wins = mc > m1
            orv[s] = jnp.where(c_wins, rc, r1)
            oiv[s] = jnp.where(c_wins, ic, i1)
            return 0

        lax.fori_loop(0, VSTEPS, vec_body, 0)
        pltpu.sync_copy(orv, or_h.at[pl.ds(off, CHUNK)])
        pltpu.sync_copy(oiv, oi_h.at[pl.ds(off, CHUNK)])
        return 0

    lax.fori_loop(0, CHUNKS, chunk_body, 0)


def kernel(Fea_A_r, Fea_B_r, Fea_C_r, Fea_A_i, Fea_B_i, Fea_C_i):
    shape = Fea_A_r.shape
    flat = lambda x: x.reshape(-1)
    out_r, out_i = _sc_max_fusion(
        flat(Fea_A_r), flat(Fea_B_r), flat(Fea_C_r),
        flat(Fea_A_i), flat(Fea_B_i), flat(Fea_C_i),
    )
    return out_r.reshape(shape), out_i.reshape(shape)


# speedup vs baseline: 7.6777x; 1.1776x over previous
import jax
import jax.numpy as jnp
from jax import lax
from jax.experimental import pallas as pl
from jax.experimental.pallas import tpu as pltpu
from jax.experimental.pallas import tpu_sc as plsc
import functools

N_TOTAL = 16 * 192 * 56 * 56
NC, NS, L = 2, 16, 16
NW = NC * NS
PER_W = N_TOTAL // NW
CHUNK = 6272
CHUNKS = PER_W // CHUNK
VSTEPS = CHUNK // L
NBUF = 2

_mesh = plsc.VectorSubcoreMesh(
    core_axis_name="c", subcore_axis_name="s", num_cores=NC, num_subcores=NS
)


@functools.partial(
    pl.kernel,
    out_type=(
        jax.ShapeDtypeStruct((N_TOTAL,), jnp.float32),
        jax.ShapeDtypeStruct((N_TOTAL,), jnp.float32),
    ),
    mesh=_mesh,
    scratch_types=[
        pltpu.VMEM((NBUF, CHUNK), jnp.float32),
        pltpu.VMEM((NBUF, CHUNK), jnp.float32),
        pltpu.VMEM((NBUF, CHUNK), jnp.float32),
        pltpu.VMEM((NBUF, CHUNK), jnp.float32),
        pltpu.VMEM((NBUF, CHUNK), jnp.float32),
        pltpu.VMEM((NBUF, CHUNK), jnp.float32),
        pltpu.VMEM((NBUF, CHUNK), jnp.float32),
        pltpu.VMEM((NBUF, CHUNK), jnp.float32),
        pltpu.SemaphoreType.DMA,
        pltpu.SemaphoreType.DMA,
        pltpu.SemaphoreType.DMA,
        pltpu.SemaphoreType.DMA,
    ],
)
def _sc_max_fusion(ar_h, br_h, cr_h, ai_h, bi_h, ci_h, or_h, oi_h,
                   ar_v, br_v, cr_v, ai_v, bi_v, ci_v, orv, oiv,
                   in_sem0, in_sem1, out_sem0, out_sem1):
    wid = lax.axis_index("s") * NC + lax.axis_index("c")
    base = wid * PER_W
    in_sems = (in_sem0, in_sem1)
    out_sems = (out_sem0, out_sem1)
    in_refs = (ar_v, br_v, cr_v, ai_v, bi_v, ci_v)
    in_hbm = (ar_h, br_h, cr_h, ai_h, bi_h, ci_h)

    def issue_in(k, b):
        off = pl.multiple_of(base + k * CHUNK, CHUNK)
        for h, v in zip(in_hbm, in_refs):
            pltpu.async_copy(h.at[pl.ds(off, CHUNK)], v.at[b], in_sems[b])

    def wait_in(b):
        for h, v in zip(in_hbm, in_refs):
            pltpu.make_async_copy(h.at[pl.ds(0, CHUNK)], v.at[b], in_sems[b]).wait()

    def issue_out(k, b):
        off = pl.multiple_of(base + k * CHUNK, CHUNK)
        pltpu.async_copy(orv.at[b], or_h.at[pl.ds(off, CHUNK)], out_sems[b])
        pltpu.async_copy(oiv.at[b], oi_h.at[pl.ds(off, CHUNK)], out_sems[b])

    def wait_out(b):
        pltpu.make_async_copy(orv.at[b], or_h.at[pl.ds(0, CHUNK)], out_sems[b]).wait()
        pltpu.make_async_copy(oiv.at[b], oi_h.at[pl.ds(0, CHUNK)], out_sems[b]).wait()

    issue_in(0, 0)
    issue_in(1, 1)

    def step(i, _):
        for b in range(NBUF):
            k = i * NBUF + b
            wait_in(b)

            @pl.when(k >= NBUF)
            def _():
                wait_out(b)

            @plsc.parallel_loop(0, VSTEPS, unroll=4)
            def _(j):
                s = pl.ds(j * L, L)
                ra = ar_v[b, s]
                ia = ai_v[b, s]
                rb = br_v[b, s]
                ib = bi_v[b, s]
                rc = cr_v[b, s]
                ic = ci_v[b, s]
                ma = ra * ra + ia * ia
                mb = rb * rb + ib * ib
                mc = rc * rc + ic * ic
                b_wins = mb > ma
                r1 = jnp.where(b_wins, rb, ra)
                i1 = jnp.where(b_wins, ib, ia)
                m1 = jnp.maximum(ma, mb)
                c_wins = mc > m1
                orv[b, s] = jnp.where(c_wins, rc, r1)
                oiv[b, s] = jnp.where(c_wins, ic, i1)

            @pl.when(k + NBUF < CHUNKS)
            def _():
                issue_in(k + NBUF, b)

            issue_out(k, b)
        return 0

    lax.fori_loop(0, CHUNKS // NBUF, step, 0)
    wait_out(0)
    wait_out(1)


def kernel(Fea_A_r, Fea_B_r, Fea_C_r, Fea_A_i, Fea_B_i, Fea_C_i):
    shape = Fea_A_r.shape
    flat = lambda x: x.reshape(-1)
    out_r, out_i = _sc_max_fusion(
        flat(Fea_A_r), flat(Fea_B_r), flat(Fea_C_r),
        flat(Fea_A_i), flat(Fea_B_i), flat(Fea_C_i),
    )
    return out_r.reshape(shape), out_i.reshape(shape)


# speedup vs baseline: 12.4607x; 1.6230x over previous
import jax
import jax.numpy as jnp
from jax.experimental import pallas as pl
from jax.experimental.pallas import tpu as pltpu
import functools

P, Q = 56, 56
ROWS = 16 * 192
BLK = 24


def _body(ar, br, cr, ai, bi, ci, o_r, o_i):
    ra = ar[...]
    ia = ai[...]
    rb = br[...]
    ib = bi[...]
    rc = cr[...]
    ic = ci[...]
    ma = ra * ra + ia * ia
    mb = rb * rb + ib * ib
    mc = rc * rc + ic * ic
    b_wins = mb > ma
    r1 = jnp.where(b_wins, rb, ra)
    i1 = jnp.where(b_wins, ib, ia)
    m1 = jnp.maximum(ma, mb)
    c_wins = mc > m1
    o_r[...] = jnp.where(c_wins, rc, r1)
    o_i[...] = jnp.where(c_wins, ic, i1)


@jax.jit
def _tc_max_fusion(ar, br, cr, ai, bi, ci):
    spec = pl.BlockSpec((BLK, P, Q), lambda i: (i, 0, 0))
    return pl.pallas_call(
        _body,
        grid=(ROWS // BLK,),
        in_specs=[spec] * 6,
        out_specs=[spec] * 2,
        out_shape=[jax.ShapeDtypeStruct((ROWS, P, Q), jnp.float32)] * 2,
    )(ar, br, cr, ai, bi, ci)


def kernel(Fea_A_r, Fea_B_r, Fea_C_r, Fea_A_i, Fea_B_i, Fea_C_i):
    shape = Fea_A_r.shape
    f = lambda x: x.reshape(ROWS, P, Q)
    out_r, out_i = _tc_max_fusion(
        f(Fea_A_r), f(Fea_B_r), f(Fea_C_r),
        f(Fea_A_i), f(Fea_B_i), f(Fea_C_i),
    )
    return out_r.reshape(shape), out_i.reshape(shape)
